# baseline (speedup 1.0000x reference)
import jax
import jax.numpy as jnp
from jax import lax
from jax.experimental import pallas as pl
from jax.experimental.pallas import tpu as pltpu
from jax.experimental.pallas import tpu_sc as plsc

_D = 64
_NC = 2
_NS = 16
_NW = _NC * _NS
_CHUNK = 1024


def _emb_body(x_hbm, table_hbm, out_hbm, idx_v, rows_v, sem):
    wid = lax.axis_index("s") * _NC + lax.axis_index("c")
    b_per_w = x_hbm.shape[0] // _NW
    n_chunks = b_per_w // _CHUNK
    base = wid * b_per_w

    def body(i, carry):
        off = base + i * _CHUNK
        pltpu.sync_copy(x_hbm.at[pl.ds(off, _CHUNK)], idx_v)
        pltpu.async_copy(table_hbm.at[idx_v], rows_v, sem).wait()
        pltpu.sync_copy(rows_v, out_hbm.at[pl.ds(off, _CHUNK)])
        return carry

    lax.fori_loop(0, n_chunks, body, 0)


def kernel(x, table):
    b, s = x.shape
    n = b * s
    xf = x.reshape(n).astype(jnp.int32)
    mesh = plsc.VectorSubcoreMesh(core_axis_name="c", subcore_axis_name="s")
    out = pl.kernel(
        _emb_body,
        out_type=jax.ShapeDtypeStruct((n, _D), jnp.float32),
        mesh=mesh,
        scratch_types=[
            pltpu.VMEM((_CHUNK,), jnp.int32),
            pltpu.VMEM((_CHUNK, _D), jnp.float32),
            pltpu.SemaphoreType.DMA,
        ],
        compiler_params=pltpu.CompilerParams(use_tc_tiling_on_sc=False),
    )(xf, table)
    return out.reshape(b, s, _D)


# speedup vs baseline: 1.0255x; 1.0255x over previous
import jax
import jax.numpy as jnp
from jax import lax
from jax.experimental import pallas as pl
from jax.experimental.pallas import tpu as pltpu
from jax.experimental.pallas import tpu_sc as plsc

_D = 64
_NC = 2
_NS = 16
_NW = _NC * _NS
_CHUNK = 800
_SB = 8


def _emb_body(x_hbm, table_hbm, out_hbm, idx_v, rows_v,
              s_i0, s_i1, s_g0, s_g1, s_o0, s_o1):
    wid = lax.axis_index("s") * _NC + lax.axis_index("c")
    n_chunks_total = x_hbm.shape[0]
    chunks_per_w = n_chunks_total // _NW
    n_sb = chunks_per_w // _SB
    w_chunk0 = wid * chunks_per_w

    idx_sems = (s_i0, s_i1)
    g_sems = (s_g0, s_g1)
    o_sems = (s_o0, s_o1)

    def start_idx(sb, buf):
        return pltpu.async_copy(
            x_hbm.at[pl.ds(w_chunk0 + sb * _SB, _SB), :],
            idx_v.at[buf], idx_sems[buf])

    def wait_idx(buf):
        pltpu.make_async_copy(
            x_hbm.at[pl.ds(w_chunk0, _SB), :],
            idx_v.at[buf], idx_sems[buf]).wait()

    def inner(chunk0, buf):
        g = [None, None]
        o = [None, None]

        def start_gather(c, rb):
            return pltpu.async_copy(
                table_hbm.at[idx_v.at[buf, c]], rows_v.at[rb], g_sems[rb])

        g[0] = start_gather(0, 0)
        for c in range(_SB):
            rb = c & 1
            g[rb].wait()
            if c + 1 < _SB:
                nb = (c + 1) & 1
                if o[nb] is not None:
                    o[nb].wait()
                g[nb] = start_gather(c + 1, nb)
            o[rb] = pltpu.async_copy(
                rows_v.at[rb],
                out_hbm.at[pl.ds((chunk0 + c) * _CHUNK, _CHUNK)],
                o_sems[rb])
        for rb in (0, 1):
            if o[rb] is not None:
                o[rb].wait()

    start_idx(0, 0)

    def body(j, carry):
        s0 = 2 * j
        s1 = 2 * j + 1
        wait_idx(0)
        start_idx(s1, 1)
        inner(w_chunk0 + s0 * _SB, 0)
        wait_idx(1)
        start_idx(jnp.minimum(s1 + 1, n_sb - 1), 0)
        inner(w_chunk0 + s1 * _SB, 1)
        return carry

    lax.fori_loop(0, n_sb // 2, body, 0)
    wait_idx(0)


def kernel(x, table):
    b, s = x.shape
    n = b * s
    xc = x.reshape(n // _CHUNK, _CHUNK).astype(jnp.int32)
    mesh = plsc.VectorSubcoreMesh(core_axis_name="c", subcore_axis_name="s")
    out = pl.kernel(
        _emb_body,
        out_type=jax.ShapeDtypeStruct((n, _D), jnp.float32),
        mesh=mesh,
        scratch_types=[
            pltpu.VMEM((2, _SB, _CHUNK), jnp.int32),
            pltpu.VMEM((2, _CHUNK, _D), jnp.float32),
            pltpu.SemaphoreType.DMA,
            pltpu.SemaphoreType.DMA,
            pltpu.SemaphoreType.DMA,
            pltpu.SemaphoreType.DMA,
            pltpu.SemaphoreType.DMA,
            pltpu.SemaphoreType.DMA,
        ],
        compiler_params=pltpu.CompilerParams(use_tc_tiling_on_sc=False),
    )(xc, table)
    return out.reshape(b, s, _D)
